# baseline (speedup 1.0000x reference)
import functools

import jax
import jax.numpy as jnp
from jax import lax
from jax.experimental import pallas as pl
from jax.experimental.pallas import tpu as pltpu
from jax.experimental.pallas import tpu_sc as plsc

N, E, D_FEAT, D_EDGE, EMB, H = 10000, 160000, 256, 16, 16, 1024
NSLICE = 8
EPAD = 163840
NWORK = 32
EDGES_PER_W = EPAD // NWORK
CHUNK = 128
NCHUNK = EDGES_PER_W // CHUNK
NACC = 10112
ROWS_PER_TILE = NACC // 16
ZCHUNKS = [128, 128, 128, 128, 120]


def _leaky(v):
    return jnp.where(v >= 0, v, 0.01 * v)


def _h_body(x_ref, enc_ref, wx_ref, we_ref, b_ref, o_ref, ot_ref):
    acc = lax.dot_general(
        x_ref[...], wx_ref[...], (((1,), (0,)), ((), ())),
        preferred_element_type=jnp.float32)
    acc += lax.dot_general(
        enc_ref[...], we_ref[...], (((1,), (0,)), ((), ())),
        preferred_element_type=jnp.float32)
    acc += b_ref[...]
    acc = _leaky(acc)
    o_ref[...] = acc
    for p in range(NSLICE):
        ot_ref[p] = acc[:, p * 128:(p + 1) * 128]


def _compute_h(xp, encp, W_in, b_in):
    NB = 1000
    grid = (N // NB,)
    return pl.pallas_call(
        _h_body,
        grid=grid,
        in_specs=[
            pl.BlockSpec((NB, D_FEAT), lambda i: (i, 0)),
            pl.BlockSpec((NB, EMB), lambda i: (i, 0)),
            pl.BlockSpec((D_FEAT, H), lambda i: (0, 0)),
            pl.BlockSpec((EMB, H), lambda i: (0, 0)),
            pl.BlockSpec((1, H), lambda i: (0, 0)),
        ],
        out_specs=[
            pl.BlockSpec((NB, H), lambda i: (i, 0)),
            pl.BlockSpec((NSLICE, NB, 128), lambda i: (0, i, 0)),
        ],
        out_shape=[
            jax.ShapeDtypeStruct((N, H), jnp.float32),
            jax.ShapeDtypeStruct((NSLICE, N, 128), jnp.float32),
        ],
    )(xp, encp, W_in[:D_FEAT], W_in[D_FEAT:], b_in[None, :])


def _ee_body(a_ref, w_ref, b_ref, *o_refs):
    EB = a_ref.shape[0]
    acc = lax.dot_general(
        a_ref[...], w_ref[...], (((1,), (0,)), ((), ())),
        preferred_element_type=jnp.float32)
    acc += b_ref[...]
    acc = _leaky(acc)
    rowid = pl.program_id(0) * EB + lax.broadcasted_iota(jnp.int32, (EB, 1), 0)
    acc = jnp.where(rowid < E, acc, -1e30)
    for p in range(NSLICE):
        o_refs[p][...] = acc[:, p * 128:(p + 1) * 128]


def _compute_edge_emb(edge_attr_p, W_edge, b_edge):
    EB = 4096
    grid = (EPAD // EB,)
    return pl.pallas_call(
        _ee_body,
        grid=grid,
        in_specs=[
            pl.BlockSpec((EB, D_EDGE), lambda i: (i, 0)),
            pl.BlockSpec((D_EDGE, H), lambda i: (0, 0)),
            pl.BlockSpec((1, H), lambda i: (0, 0)),
        ],
        out_specs=[pl.BlockSpec((EB, 128), lambda i: (i, 0))] * NSLICE,
        out_shape=[jax.ShapeDtypeStruct((EPAD, 128), jnp.float32)] * NSLICE,
    )(edge_attr_p, W_edge, b_edge[None, :])


def _sc_edge_body(h_view, srcp, dstp, *rest):
    ee_refs = rest[:NSLICE]
    agg_refs = rest[NSLICE:2 * NSLICE]
    sidx, dsti, gidx, eebuf, rows, zbuf, shared, sem = rest[2 * NSLICE:]

    cid = lax.axis_index("c")
    sid = lax.axis_index("s")
    wid = cid * 16 + sid
    edge0 = wid * EDGES_PER_W
    row0 = sid * ROWS_PER_TILE

    def _zero_row(r, carry):
        for c in range(8):
            zbuf[r, pl.ds(c * 16, 16)] = jnp.zeros((16,), jnp.float32)
        return carry
    lax.fori_loop(0, CHUNK, _zero_row, 0)

    for p in range(NSLICE):
        off = 0
        for zc in ZCHUNKS:
            pltpu.sync_copy(
                zbuf.at[pl.ds(0, zc)], shared.at[pl.ds(row0 + off, zc)])
            off += zc
        plsc.subcore_barrier()

        def _chunk(c, carry, p=p):
            base = pl.multiple_of(edge0 + c * CHUNK, CHUNK)
            pltpu.sync_copy(srcp.at[pl.ds(base, CHUNK)], sidx)
            pltpu.sync_copy(dstp.at[pl.ds(base, CHUNK)], dsti)
            for i in range(CHUNK // 16):
                gidx[pl.ds(i * 16, 16)] = sidx[pl.ds(i * 16, 16)] + p * N
            pltpu.async_copy(h_view.at[gidx], rows, sem).wait()
            pltpu.sync_copy(ee_refs[p].at[pl.ds(base, CHUNK)], eebuf)

            def _row(r, c2):
                for cc in range(8):
                    sl = pl.ds(cc * 16, 16)
                    rows[r, sl] = jnp.maximum(rows[r, sl] + eebuf[r, sl], 0.0)
                return c2
            lax.fori_loop(0, CHUNK, _row, 0)

            pltpu.sync_copy(rows, shared.at[dsti], add=True)
            return carry
        lax.fori_loop(0, NCHUNK, _chunk, 0)

        plsc.subcore_barrier()
        pltpu.sync_copy(
            shared.at[pl.ds(row0, ROWS_PER_TILE)],
            agg_refs[p].at[cid, pl.ds(row0, ROWS_PER_TILE)])


def _sc_edge(h_t, srcp, dstp, ee_slices):
    h_view = h_t.reshape(NSLICE * N, 128)
    mesh = plsc.VectorSubcoreMesh(core_axis_name="c", subcore_axis_name="s")
    f = pl.kernel(
        _sc_edge_body,
        out_type=[jax.ShapeDtypeStruct((2, NACC, 128), jnp.float32)] * NSLICE,
        mesh=mesh,
        scratch_types=[
            pltpu.VMEM((CHUNK,), jnp.int32),
            pltpu.VMEM((CHUNK,), jnp.int32),
            pltpu.VMEM((CHUNK,), jnp.int32),
            pltpu.VMEM((CHUNK, 128), jnp.float32),
            pltpu.VMEM((CHUNK, 128), jnp.float32),
            pltpu.VMEM((CHUNK, 128), jnp.float32),
            pltpu.VMEM_SHARED((NACC, 128), jnp.float32),
            pltpu.SemaphoreType.DMA,
        ],
    )
    return f(h_view, srcp, dstp, *ee_slices)


def _mlp_body(h_ref, *rest):
    agg_refs = rest[:NSLICE]
    w1_ref, b1_ref, w2_ref, b2_ref, g_ref, be_ref, o_ref = rest[NSLICE:]
    agg = jnp.concatenate([a[0] + a[1] for a in agg_refs], axis=-1)
    v = h_ref[...] + agg
    v = _leaky(lax.dot_general(
        v, w1_ref[...], (((1,), (0,)), ((), ())),
        preferred_element_type=jnp.float32) + b1_ref[...])
    v = lax.dot_general(
        v, w2_ref[...], (((1,), (0,)), ((), ())),
        preferred_element_type=jnp.float32) + b2_ref[...]
    v = _leaky(v)
    mu = jnp.mean(v, axis=-1, keepdims=True)
    var = jnp.mean((v - mu) ** 2, axis=-1, keepdims=True)
    o_ref[...] = (v - mu) * lax.rsqrt(var + 1e-5) * g_ref[...] + be_ref[...]


def _compute_out(h, agg_slices, W1, b1, W2, b2, ln_g, ln_b):
    NB = 1000
    grid = (N // NB,)
    return pl.pallas_call(
        _mlp_body,
        grid=grid,
        in_specs=(
            [pl.BlockSpec((NB, H), lambda i: (i, 0))]
            + [pl.BlockSpec((2, NB, 128), lambda i: (0, i, 0))] * NSLICE
            + [
                pl.BlockSpec((H, H), lambda i: (0, 0)),
                pl.BlockSpec((1, H), lambda i: (0, 0)),
                pl.BlockSpec((H, H), lambda i: (0, 0)),
                pl.BlockSpec((1, H), lambda i: (0, 0)),
                pl.BlockSpec((1, H), lambda i: (0, 0)),
                pl.BlockSpec((1, H), lambda i: (0, 0)),
            ]
        ),
        out_specs=pl.BlockSpec((NB, H), lambda i: (i, 0)),
        out_shape=jax.ShapeDtypeStruct((N, H), jnp.float32),
    )(h, *agg_slices, W1, b1[None, :], W2, b2[None, :],
      ln_g[None, :], ln_b[None, :])


def kernel(x, edge_index, edge_attr, group_emb, W_in, b_in, W_edge, b_edge,
           W1, b1, W2, b2, ln_g, ln_b):
    enc_head = jnp.repeat(group_emb, 16, axis=0)
    encp = jnp.concatenate(
        [enc_head, jnp.zeros((N - 128, EMB), jnp.float32)], axis=0)
    xp = x

    srcp = jnp.concatenate(
        [edge_index[0], jnp.zeros((EPAD - E,), jnp.int32)])
    dstp = jnp.concatenate(
        [edge_index[1], jnp.zeros((EPAD - E,), jnp.int32)])
    eap = jnp.concatenate(
        [edge_attr, jnp.zeros((EPAD - E, D_EDGE), jnp.float32)], axis=0)

    h, h_t = _compute_h(xp, encp, W_in, b_in)
    ee_slices = _compute_edge_emb(eap, W_edge, b_edge)

    agg_slices = _sc_edge(h_t, srcp, dstp, ee_slices)

    out = _compute_out(h, agg_slices, W1, b1, W2, b2, ln_g, ln_b)
    return (out[:N], edge_attr)


# speedup vs baseline: 1.4347x; 1.4347x over previous
import functools

import jax
import jax.numpy as jnp
from jax import lax
from jax.experimental import pallas as pl
from jax.experimental.pallas import tpu as pltpu
from jax.experimental.pallas import tpu_sc as plsc

N, E, D_FEAT, D_EDGE, EMB, H = 10000, 160000, 256, 16, 16, 1024
NSLICE = 8
EPAD = 163840
NWORK = 32
EDGES_PER_W = EPAD // NWORK
CHUNK = 64
NCHUNK = EDGES_PER_W // CHUNK
NACC = 10112
ROWS_PER_TILE = NACC // 16
ZCHUNKS = [64] * 9 + [56]


def _leaky(v):
    return jnp.where(v >= 0, v, 0.01 * v)


def _h_body(x_ref, enc_ref, wx_ref, we_ref, b_ref, o_ref, *os_refs):
    acc = lax.dot_general(
        x_ref[...], wx_ref[...], (((1,), (0,)), ((), ())),
        preferred_element_type=jnp.float32)
    acc += lax.dot_general(
        enc_ref[...], we_ref[...], (((1,), (0,)), ((), ())),
        preferred_element_type=jnp.float32)
    acc += b_ref[...]
    acc = _leaky(acc)
    o_ref[...] = acc
    for p in range(NSLICE):
        os_refs[p][...] = acc[:, p * 128:(p + 1) * 128]


def _compute_h(xp, encp, W_in, b_in):
    NB = 1000
    grid = (N // NB,)
    return pl.pallas_call(
        _h_body,
        grid=grid,
        in_specs=[
            pl.BlockSpec((NB, D_FEAT), lambda i: (i, 0)),
            pl.BlockSpec((NB, EMB), lambda i: (i, 0)),
            pl.BlockSpec((D_FEAT, H), lambda i: (0, 0)),
            pl.BlockSpec((EMB, H), lambda i: (0, 0)),
            pl.BlockSpec((1, H), lambda i: (0, 0)),
        ],
        out_specs=([pl.BlockSpec((NB, H), lambda i: (i, 0))]
                   + [pl.BlockSpec((NB, 128), lambda i: (i, 0))] * NSLICE),
        out_shape=([jax.ShapeDtypeStruct((N, H), jnp.float32)]
                   + [jax.ShapeDtypeStruct((N, 128), jnp.float32)] * NSLICE),
    )(xp, encp, W_in[:D_FEAT], W_in[D_FEAT:], b_in[None, :])


def _ee_body(a_ref, w_ref, b_ref, *o_refs):
    EB = a_ref.shape[0]
    acc = lax.dot_general(
        a_ref[...], w_ref[...], (((1,), (0,)), ((), ())),
        preferred_element_type=jnp.float32)
    acc += b_ref[...]
    acc = _leaky(acc)
    rowid = pl.program_id(0) * EB + lax.broadcasted_iota(jnp.int32, (EB, 1), 0)
    acc = jnp.where(rowid < E, acc, -1e30)
    for p in range(NSLICE):
        o_refs[p][...] = acc[:, p * 128:(p + 1) * 128]


def _compute_edge_emb(edge_attr_p, W_edge, b_edge):
    EB = 4096
    grid = (EPAD // EB,)
    return pl.pallas_call(
        _ee_body,
        grid=grid,
        in_specs=[
            pl.BlockSpec((EB, D_EDGE), lambda i: (i, 0)),
            pl.BlockSpec((D_EDGE, H), lambda i: (0, 0)),
            pl.BlockSpec((1, H), lambda i: (0, 0)),
        ],
        out_specs=[pl.BlockSpec((EB, 128), lambda i: (i, 0))] * NSLICE,
        out_shape=[jax.ShapeDtypeStruct((EPAD, 128), jnp.float32)] * NSLICE,
    )(edge_attr_p, W_edge, b_edge[None, :])


def _sc_edge_body(srcp2d, dstp2d, *rest):
    hs_refs = rest[:NSLICE]
    ee_refs = rest[NSLICE:2 * NSLICE]
    agg_refs = rest[2 * NSLICE:3 * NSLICE]
    scr = rest[3 * NSLICE:]
    sibufs = scr[0:2]
    dibufs = scr[2:6]
    eebufs = scr[6:8]
    rbufs = scr[8:12]
    shared = scr[12]
    semis, semid = scr[13:15], scr[15:19]
    semes, semgs, semss = scr[19:21], scr[21:25], scr[25:29]

    cid = lax.axis_index("c")
    sid = lax.axis_index("s")
    wid = cid * 16 + sid
    crow0 = wid * NCHUNK
    edge0 = wid * EDGES_PER_W
    row0 = sid * ROWS_PER_TILE

    def _ld_idx(c, b, d):
        pltpu.async_copy(srcp2d.at[crow0 + c], sibufs[b], semis[b])
        pltpu.async_copy(dstp2d.at[crow0 + c], dibufs[d], semid[d])

    def _wt_idx(b, d):
        pltpu.make_async_copy(srcp2d.at[crow0], sibufs[b], semis[b]).wait()
        pltpu.make_async_copy(dstp2d.at[crow0], dibufs[d], semid[d]).wait()

    def _ld_data(c, b2, ri, p):
        pltpu.async_copy(hs_refs[p].at[sibufs[b2]], rbufs[ri], semgs[ri])
        base = pl.multiple_of(edge0 + c * CHUNK, CHUNK)
        pltpu.async_copy(ee_refs[p].at[pl.ds(base, CHUNK)], eebufs[b2],
                         semes[b2])

    def _wt_data(b2, ri, p):
        pltpu.make_async_copy(
            hs_refs[p].at[sibufs[b2]], rbufs[ri], semgs[ri]).wait()
        pltpu.make_async_copy(
            ee_refs[p].at[pl.ds(edge0, CHUNK)], eebufs[b2], semes[b2]).wait()

    def _wt_scat(k):
        pltpu.make_async_copy(
            rbufs[k], shared.at[dibufs[k]], semss[k]).wait()

    def _compute_scatter(b2, k):
        rows, eeb = rbufs[k], eebufs[b2]

        def _row(r, c2):
            for cc in range(8):
                sl = pl.ds(cc * 16, 16)
                rows[r, sl] = jnp.maximum(rows[r, sl] + eeb[r, sl], 0.0)
            return c2
        lax.fori_loop(0, CHUNK, _row, 0)
        pltpu.async_copy(rows, shared.at[dibufs[k]], semss[k], add=True)

    def _step(c, j, p, first4):
        b2 = j % 2
        nb2 = (j + 1) % 2
        if not (first4 and j < 2):
            _wt_scat((j + 2) % 4)
        _wt_idx(nb2, (j + 1) % 4)
        _ld_data(c + 1, nb2, (j + 1) % 4, p)
        _wt_data(b2, j, p)
        _ld_idx(c + 2, b2, (j + 2) % 4)
        _compute_scatter(b2, j)

    for p in range(NSLICE):
        def _zero_row(r, carry):
            for c in range(8):
                rbufs[0][r, pl.ds(c * 16, 16)] = jnp.zeros((16,), jnp.float32)
            return carry
        lax.fori_loop(0, CHUNK, _zero_row, 0)
        off = 0
        for zc in ZCHUNKS:
            pltpu.sync_copy(
                rbufs[0].at[pl.ds(0, zc)], shared.at[pl.ds(row0 + off, zc)])
            off += zc
        plsc.subcore_barrier()

        _ld_idx(0, 0, 0)
        _ld_idx(1, 1, 1)
        _wt_idx(0, 0)
        _ld_data(0, 0, 0, p)

        for j in range(4):
            _step(j, j, p, True)

        def _quad(g, carry, p=p):
            c0 = g * 4
            for j in range(4):
                _step(c0 + j, j, p, False)
            return carry
        lax.fori_loop(1, NCHUNK // 4 - 1, _quad, 0)

        cl = NCHUNK - 4
        for j in range(4):
            c = cl + j
            b2 = j % 2
            nb2 = (j + 1) % 2
            _wt_scat((j + 2) % 4)
            if j < 3:
                _wt_idx(nb2, (j + 1) % 4)
                _ld_data(c + 1, nb2, (j + 1) % 4, p)
            _wt_data(b2, j, p)
            if j < 2:
                _ld_idx(c + 2, b2, (j + 2) % 4)
            _compute_scatter(b2, j)

        _wt_scat(2)
        _wt_scat(3)

        plsc.subcore_barrier()
        pltpu.sync_copy(
            shared.at[pl.ds(row0, ROWS_PER_TILE)],
            agg_refs[p].at[cid, pl.ds(row0, ROWS_PER_TILE)])


def _sc_edge(h_slices, srcp, dstp, ee_slices):
    srcp2d = srcp.reshape(EPAD // CHUNK, CHUNK)
    dstp2d = dstp.reshape(EPAD // CHUNK, CHUNK)
    mesh = plsc.VectorSubcoreMesh(core_axis_name="c", subcore_axis_name="s")
    f = pl.kernel(
        _sc_edge_body,
        out_type=[jax.ShapeDtypeStruct((2, NACC, 128), jnp.float32)] * NSLICE,
        mesh=mesh,
        scratch_types=(
            [pltpu.VMEM((CHUNK,), jnp.int32)] * 6
            + [pltpu.VMEM((CHUNK, 128), jnp.float32)] * 2
            + [pltpu.VMEM((CHUNK, 128), jnp.float32)] * 4
            + [pltpu.VMEM_SHARED((NACC, 128), jnp.float32)]
            + [pltpu.SemaphoreType.DMA] * 16
        ),
    )
    return f(srcp2d, dstp2d, *h_slices, *ee_slices)


def _mlp_body(h_ref, *rest):
    agg_refs = rest[:NSLICE]
    w1_ref, b1_ref, w2_ref, b2_ref, g_ref, be_ref, o_ref = rest[NSLICE:]
    agg = jnp.concatenate([a[0] + a[1] for a in agg_refs], axis=-1)
    v = h_ref[...] + agg
    v = _leaky(lax.dot_general(
        v, w1_ref[...], (((1,), (0,)), ((), ())),
        preferred_element_type=jnp.float32) + b1_ref[...])
    v = lax.dot_general(
        v, w2_ref[...], (((1,), (0,)), ((), ())),
        preferred_element_type=jnp.float32) + b2_ref[...]
    v = _leaky(v)
    mu = jnp.mean(v, axis=-1, keepdims=True)
    var = jnp.mean((v - mu) ** 2, axis=-1, keepdims=True)
    o_ref[...] = (v - mu) * lax.rsqrt(var + 1e-5) * g_ref[...] + be_ref[...]


def _compute_out(h, agg_slices, W1, b1, W2, b2, ln_g, ln_b):
    NB = 1000
    grid = (N // NB,)
    return pl.pallas_call(
        _mlp_body,
        grid=grid,
        in_specs=(
            [pl.BlockSpec((NB, H), lambda i: (i, 0))]
            + [pl.BlockSpec((2, NB, 128), lambda i: (0, i, 0))] * NSLICE
            + [
                pl.BlockSpec((H, H), lambda i: (0, 0)),
                pl.BlockSpec((1, H), lambda i: (0, 0)),
                pl.BlockSpec((H, H), lambda i: (0, 0)),
                pl.BlockSpec((1, H), lambda i: (0, 0)),
                pl.BlockSpec((1, H), lambda i: (0, 0)),
                pl.BlockSpec((1, H), lambda i: (0, 0)),
            ]
        ),
        out_specs=pl.BlockSpec((NB, H), lambda i: (i, 0)),
        out_shape=jax.ShapeDtypeStruct((N, H), jnp.float32),
    )(h, *agg_slices, W1, b1[None, :], W2, b2[None, :],
      ln_g[None, :], ln_b[None, :])


def kernel(x, edge_index, edge_attr, group_emb, W_in, b_in, W_edge, b_edge,
           W1, b1, W2, b2, ln_g, ln_b):
    enc_head = jnp.repeat(group_emb, 16, axis=0)
    encp = jnp.concatenate(
        [enc_head, jnp.zeros((N - 128, EMB), jnp.float32)], axis=0)
    xp = x

    srcp = jnp.concatenate(
        [edge_index[0], jnp.zeros((EPAD - E,), jnp.int32)])
    dstp = jnp.concatenate(
        [edge_index[1], jnp.zeros((EPAD - E,), jnp.int32)])
    eap = jnp.concatenate(
        [edge_attr, jnp.zeros((EPAD - E, D_EDGE), jnp.float32)], axis=0)

    h, *h_slices = _compute_h(xp, encp, W_in, b_in)
    ee_slices = _compute_edge_emb(eap, W_edge, b_edge)

    agg_slices = _sc_edge(h_slices, srcp, dstp, ee_slices)

    out = _compute_out(h, agg_slices, W1, b1, W2, b2, ln_g, ln_b)
    return (out[:N], edge_attr)
